# Initial kernel scaffold; baseline (speedup 1.0000x reference)
import jax
import jax.numpy as jnp
from jax.experimental import pallas as pl


def kernel(x, e, edge_index, node_W, node_b, edge_W, edge_b, Wu, bu, Wv, bv, Wa, ba, Wb, bb, Wc, bc, ln_x_g, ln_x_b, ln_e_g, ln_e_b, out_W, out_b):
    raise NotImplementedError("write your pallas kernel here")


# speedup vs baseline: 1.0221x; 1.0221x over previous
import functools

import jax
import jax.numpy as jnp
from jax import lax
from jax.experimental import pallas as pl
from jax.experimental.pallas import tpu as pltpu

N = 10000
E = 320000
H = 256
L = 12
EB = 1280
NB = 2000

_INTERP = False


def _ln_relu(t, g, b):
    mu = jnp.mean(t, axis=-1, keepdims=True)
    var = jnp.mean((t - mu) ** 2, axis=-1, keepdims=True)
    ln = (t - mu) * lax.rsqrt(var + 1e-5) * g + b
    return jnp.maximum(ln, 0.0)


def _embed_body(x_ref, w_ref, b_ref, o_ref):
    o_ref[...] = jnp.dot(x_ref[...], w_ref[...],
                         preferred_element_type=jnp.float32) + b_ref[...]


def _embed(x, node_W, node_b):
    return pl.pallas_call(
        _embed_body,
        grid=(N // NB,),
        in_specs=[pl.BlockSpec((NB, 2), lambda i: (i, 0)),
                  pl.BlockSpec((2, H), lambda i: (0, 0)),
                  pl.BlockSpec((1, H), lambda i: (0, 0))],
        out_specs=pl.BlockSpec((NB, H), lambda i: (i, 0)),
        out_shape=jax.ShapeDtypeStruct((N, H), jnp.float32),
        interpret=_INTERP,
    )(x, node_W, node_b.reshape(1, H))


def _node_mm_body(h_ref, w4_ref, b4_ref, u_ref, v_ref, a_ref, b_ref):
    hb = h_ref[...]
    w4 = w4_ref[...]
    outs = (u_ref, v_ref, a_ref, b_ref)
    for j in range(4):
        outs[j][...] = (jnp.dot(hb, w4[:, j * H:(j + 1) * H],
                                preferred_element_type=jnp.float32)
                        + b4_ref[0, j * H:(j + 1) * H][None, :])


def _node_mm(h, W4, b4):
    return pl.pallas_call(
        _node_mm_body,
        grid=(N // NB,),
        in_specs=[pl.BlockSpec((NB, H), lambda i: (i, 0)),
                  pl.BlockSpec((H, 4 * H), lambda i: (0, 0)),
                  pl.BlockSpec((1, 4 * H), lambda i: (0, 0))],
        out_specs=[pl.BlockSpec((NB, H), lambda i: (i, 0))] * 4,
        out_shape=[jax.ShapeDtypeStruct((N, H), jnp.float32)] * 4,
        interpret=_INTERP,
    )(h, W4, b4.reshape(1, 4 * H))


def _edge_core(f, g_ref, w_ref, wc_ref, bc_ref, lng_ref, lnb_ref):
    e_new = (g_ref[...]
             + jnp.dot(f, wc_ref[...], preferred_element_type=jnp.float32)
             + bc_ref[...])
    gates = jax.nn.sigmoid(e_new)
    f_new = f + _ln_relu(e_new, lng_ref[...], lnb_ref[...])
    p = gates * w_ref[...]
    return f_new, p


def _edge_first_body(e_ref, ew_ref, eb_ref, g_ref, w_ref, wc_ref, bc_ref,
                     lng_ref, lnb_ref, fo_ref, p_ref):
    f = e_ref[...] * ew_ref[...] + eb_ref[...]
    f_new, p = _edge_core(f, g_ref, w_ref, wc_ref, bc_ref, lng_ref, lnb_ref)
    fo_ref[...] = f_new
    p_ref[...] = p


def _edge_mid_body(f_ref, g_ref, w_ref, wc_ref, bc_ref, lng_ref, lnb_ref,
                   fo_ref, p_ref):
    f_new, p = _edge_core(f_ref[...], g_ref, w_ref, wc_ref, bc_ref,
                          lng_ref, lnb_ref)
    fo_ref[...] = f_new
    p_ref[...] = p


def _edge_last_body(f_ref, g_ref, w_ref, wc_ref, bc_ref, lng_ref, lnb_ref,
                    ow_ref, ob_ref, p_ref, eo_ref):
    f_new, p = _edge_core(f_ref[...], g_ref, w_ref, wc_ref, bc_ref,
                          lng_ref, lnb_ref)
    p_ref[...] = p
    eo_ref[...] = jnp.dot(f_new, ow_ref[...],
                          preferred_element_type=jnp.float32) + ob_ref[...]


_eb_spec = lambda: pl.BlockSpec((EB, H), lambda i: (i, 0))
_w_spec = lambda: pl.BlockSpec((H, H), lambda i: (0, 0))
_row_spec = lambda: pl.BlockSpec((1, H), lambda i: (0, 0))


def _edge_first(e2, edge_W, edge_b, G, W, wc, bc, lng, lnb):
    return pl.pallas_call(
        _edge_first_body,
        grid=(E // EB,),
        in_specs=[pl.BlockSpec((EB, 1), lambda i: (i, 0)),
                  _row_spec(), _row_spec(),
                  _eb_spec(), _eb_spec(), _w_spec(),
                  _row_spec(), _row_spec(), _row_spec()],
        out_specs=[_eb_spec(), _eb_spec()],
        out_shape=[jax.ShapeDtypeStruct((E, H), jnp.float32)] * 2,
        interpret=_INTERP,
    )(e2, edge_W, edge_b.reshape(1, H), G, W, wc,
      bc.reshape(1, H), lng.reshape(1, H), lnb.reshape(1, H))


def _edge_mid(f, G, W, wc, bc, lng, lnb):
    return pl.pallas_call(
        _edge_mid_body,
        grid=(E // EB,),
        in_specs=[_eb_spec(), _eb_spec(), _eb_spec(), _w_spec(),
                  _row_spec(), _row_spec(), _row_spec()],
        out_specs=[_eb_spec(), _eb_spec()],
        out_shape=[jax.ShapeDtypeStruct((E, H), jnp.float32)] * 2,
        interpret=_INTERP,
    )(f, G, W, wc, bc.reshape(1, H), lng.reshape(1, H), lnb.reshape(1, H))


def _edge_last(f, G, W, wc, bc, lng, lnb, out_W, out_b):
    return pl.pallas_call(
        _edge_last_body,
        grid=(E // EB,),
        in_specs=[_eb_spec(), _eb_spec(), _eb_spec(), _w_spec(),
                  _row_spec(), _row_spec(), _row_spec(),
                  pl.BlockSpec((H, 2), lambda i: (0, 0)),
                  pl.BlockSpec((1, 2), lambda i: (0, 0))],
        out_specs=[_eb_spec(), pl.BlockSpec((EB, 2), lambda i: (i, 0))],
        out_shape=[jax.ShapeDtypeStruct((E, H), jnp.float32),
                   jax.ShapeDtypeStruct((E, 2), jnp.float32)],
        interpret=_INTERP,
    )(f, G, W, wc, bc.reshape(1, H), lng.reshape(1, H), lnb.reshape(1, H),
      out_W, out_b.reshape(1, 2))


def _h_update_body(h_ref, ux_ref, agg_ref, g_ref, b_ref, o_ref):
    t = ux_ref[...] + agg_ref[...]
    o_ref[...] = h_ref[...] + _ln_relu(t, g_ref[...], b_ref[...])


def _h_update(h, Ux, agg, lng, lnb):
    return pl.pallas_call(
        _h_update_body,
        grid=(N // NB,),
        in_specs=[pl.BlockSpec((NB, H), lambda i: (i, 0))] * 3
                 + [_row_spec(), _row_spec()],
        out_specs=pl.BlockSpec((NB, H), lambda i: (i, 0)),
        out_shape=jax.ShapeDtypeStruct((N, H), jnp.float32),
        interpret=_INTERP,
    )(h, Ux, agg, lng.reshape(1, H), lnb.reshape(1, H))


def kernel(x, e, edge_index, node_W, node_b, edge_W, edge_b,
           Wu, bu, Wv, bv, Wa, ba, Wb, bb, Wc, bc,
           ln_x_g, ln_x_b, ln_e_g, ln_e_b, out_W, out_b):
    src = edge_index[0]
    dst = edge_index[1]
    W4 = jnp.concatenate([Wu, Wv, Wa, Wb], axis=2)
    b4 = jnp.concatenate([bu, bv, ba, bb], axis=1)

    h = _embed(x, node_W, node_b)
    e2 = e.reshape(E, 1)
    f = None
    e_out = None
    for i in range(L):
        Ux, Vx, Ax, Bx = _node_mm(h, W4[i], b4[i])
        G = Ax[src] + Bx[dst]
        W = Vx[src]
        if i == 0:
            f, P = _edge_first(e2, edge_W, edge_b, G, W,
                               Wc[i], bc[i], ln_e_g[i], ln_e_b[i])
        elif i < L - 1:
            f, P = _edge_mid(f, G, W, Wc[i], bc[i], ln_e_g[i], ln_e_b[i])
        else:
            P, e_out = _edge_last(f, G, W, Wc[i], bc[i],
                                  ln_e_g[i], ln_e_b[i], out_W, out_b)
        agg = jnp.zeros((N, H), jnp.float32).at[dst].add(P)
        h = _h_update(h, Ux, agg, ln_x_g[i], ln_x_b[i])
    return h, e_out


# speedup vs baseline: 2.0369x; 1.9928x over previous
import functools

import jax
import jax.numpy as jnp
from jax import lax
from jax.experimental import pallas as pl
from jax.experimental.pallas import tpu as pltpu
from jax.experimental.pallas import tpu_sc as plsc

N = 10000
E = 320000
H = 256
L = 12
EB = 1280
NB = 2000

_INTERP = False


def _ln_relu(t, g, b):
    mu = jnp.mean(t, axis=-1, keepdims=True)
    var = jnp.mean((t - mu) ** 2, axis=-1, keepdims=True)
    ln = (t - mu) * lax.rsqrt(var + 1e-5) * g + b
    return jnp.maximum(ln, 0.0)


def _embed_body(x_ref, w_ref, b_ref, o_ref):
    o_ref[...] = jnp.dot(x_ref[...], w_ref[...],
                         preferred_element_type=jnp.float32) + b_ref[...]


def _embed(x, node_W, node_b):
    return pl.pallas_call(
        _embed_body,
        grid=(N // NB,),
        in_specs=[pl.BlockSpec((NB, 2), lambda i: (i, 0)),
                  pl.BlockSpec((2, H), lambda i: (0, 0)),
                  pl.BlockSpec((1, H), lambda i: (0, 0))],
        out_specs=pl.BlockSpec((NB, H), lambda i: (i, 0)),
        out_shape=jax.ShapeDtypeStruct((N, H), jnp.float32),
        interpret=_INTERP,
    )(x, node_W, node_b.reshape(1, H))


def _node_mm_body(h_ref, w4_ref, b4_ref, u_ref, v_ref, a_ref, b_ref):
    hb = h_ref[...]
    w4 = w4_ref[...]
    outs = (u_ref, v_ref, a_ref, b_ref)
    for j in range(4):
        outs[j][...] = (jnp.dot(hb, w4[:, j * H:(j + 1) * H],
                                preferred_element_type=jnp.float32)
                        + b4_ref[0, j * H:(j + 1) * H][None, :])


def _node_mm(h, W4, b4):
    return pl.pallas_call(
        _node_mm_body,
        grid=(N // NB,),
        in_specs=[pl.BlockSpec((NB, H), lambda i: (i, 0)),
                  pl.BlockSpec((H, 4 * H), lambda i: (0, 0)),
                  pl.BlockSpec((1, 4 * H), lambda i: (0, 0))],
        out_specs=[pl.BlockSpec((NB, H), lambda i: (i, 0))] * 4,
        out_shape=[jax.ShapeDtypeStruct((N, H), jnp.float32)] * 4,
        interpret=_INTERP,
    )(h, W4, b4.reshape(1, 4 * H))


def _edge_core(f, g_ref, w_ref, wc_ref, bc_ref, lng_ref, lnb_ref):
    e_new = (g_ref[...]
             + jnp.dot(f, wc_ref[...], preferred_element_type=jnp.float32)
             + bc_ref[...])
    gates = jax.nn.sigmoid(e_new)
    f_new = f + _ln_relu(e_new, lng_ref[...], lnb_ref[...])
    p = gates * w_ref[...]
    return f_new, p


def _edge_first_body(e_ref, ew_ref, eb_ref, g_ref, w_ref, wc_ref, bc_ref,
                     lng_ref, lnb_ref, fo_ref, p_ref):
    f = e_ref[...] * ew_ref[...] + eb_ref[...]
    f_new, p = _edge_core(f, g_ref, w_ref, wc_ref, bc_ref, lng_ref, lnb_ref)
    fo_ref[...] = f_new
    p_ref[...] = p


def _edge_mid_body(f_ref, g_ref, w_ref, wc_ref, bc_ref, lng_ref, lnb_ref,
                   fo_ref, p_ref):
    f_new, p = _edge_core(f_ref[...], g_ref, w_ref, wc_ref, bc_ref,
                          lng_ref, lnb_ref)
    fo_ref[...] = f_new
    p_ref[...] = p


def _edge_last_body(f_ref, g_ref, w_ref, wc_ref, bc_ref, lng_ref, lnb_ref,
                    ow_ref, ob_ref, p_ref, eo_ref):
    f_new, p = _edge_core(f_ref[...], g_ref, w_ref, wc_ref, bc_ref,
                          lng_ref, lnb_ref)
    p_ref[...] = p
    eo_ref[...] = jnp.dot(f_new, ow_ref[...],
                          preferred_element_type=jnp.float32) + ob_ref[...]


_eb_spec = lambda: pl.BlockSpec((EB, H), lambda i: (i, 0))
_w_spec = lambda: pl.BlockSpec((H, H), lambda i: (0, 0))
_row_spec = lambda: pl.BlockSpec((1, H), lambda i: (0, 0))


def _edge_first(e2, edge_W, edge_b, G, W, wc, bc, lng, lnb):
    return pl.pallas_call(
        _edge_first_body,
        grid=(E // EB,),
        in_specs=[pl.BlockSpec((EB, 1), lambda i: (i, 0)),
                  _row_spec(), _row_spec(),
                  _eb_spec(), _eb_spec(), _w_spec(),
                  _row_spec(), _row_spec(), _row_spec()],
        out_specs=[_eb_spec(), _eb_spec()],
        out_shape=[jax.ShapeDtypeStruct((E, H), jnp.float32)] * 2,
        interpret=_INTERP,
    )(e2, edge_W, edge_b.reshape(1, H), G, W, wc,
      bc.reshape(1, H), lng.reshape(1, H), lnb.reshape(1, H))


def _edge_mid(f, G, W, wc, bc, lng, lnb):
    return pl.pallas_call(
        _edge_mid_body,
        grid=(E // EB,),
        in_specs=[_eb_spec(), _eb_spec(), _eb_spec(), _w_spec(),
                  _row_spec(), _row_spec(), _row_spec()],
        out_specs=[_eb_spec(), _eb_spec()],
        out_shape=[jax.ShapeDtypeStruct((E, H), jnp.float32)] * 2,
        interpret=_INTERP,
    )(f, G, W, wc, bc.reshape(1, H), lng.reshape(1, H), lnb.reshape(1, H))


def _edge_last(f, G, W, wc, bc, lng, lnb, out_W, out_b):
    return pl.pallas_call(
        _edge_last_body,
        grid=(E // EB,),
        in_specs=[_eb_spec(), _eb_spec(), _eb_spec(), _w_spec(),
                  _row_spec(), _row_spec(), _row_spec(),
                  pl.BlockSpec((H, 2), lambda i: (0, 0)),
                  pl.BlockSpec((1, 2), lambda i: (0, 0))],
        out_specs=[_eb_spec(), pl.BlockSpec((EB, 2), lambda i: (i, 0))],
        out_shape=[jax.ShapeDtypeStruct((E, H), jnp.float32),
                   jax.ShapeDtypeStruct((E, 2), jnp.float32)],
        interpret=_INTERP,
    )(f, G, W, wc, bc.reshape(1, H), lng.reshape(1, H), lnb.reshape(1, H),
      out_W, out_b.reshape(1, 2))


def _h_update_body(h_ref, ux_ref, agg_ref, g_ref, b_ref, o_ref):
    t = ux_ref[...] + agg_ref[...]
    o_ref[...] = h_ref[...] + _ln_relu(t, g_ref[...], b_ref[...])


def _h_update(h, Ux, agg, lng, lnb):
    return pl.pallas_call(
        _h_update_body,
        grid=(N // NB,),
        in_specs=[pl.BlockSpec((NB, H), lambda i: (i, 0))] * 3
                 + [_row_spec(), _row_spec()],
        out_specs=pl.BlockSpec((NB, H), lambda i: (i, 0)),
        out_shape=jax.ShapeDtypeStruct((N, H), jnp.float32),
        interpret=_INTERP,
    )(h, Ux, agg, lng.reshape(1, H), lnb.reshape(1, H))


NC = 2
NS = 16
NW = NC * NS

_sc_mesh = plsc.VectorSubcoreMesh(core_axis_name="c", subcore_axis_name="s")

EPW = E // NW
KG = 80


@functools.partial(
    pl.kernel, mesh=_sc_mesh,
    out_type=[jax.ShapeDtypeStruct((E, H), jnp.float32),
              jax.ShapeDtypeStruct((E, H), jnp.float32)],
    scratch_types=[pltpu.VMEM((KG,), jnp.int32),
                   pltpu.VMEM((KG,), jnp.int32),
                   pltpu.VMEM((KG, H), jnp.float32),
                   pltpu.VMEM((KG, H), jnp.float32),
                   pltpu.VMEM((KG, H), jnp.float32),
                   pltpu.SemaphoreType.DMA],
)
def _gather_sc(ax_hbm, bx_hbm, vx_hbm, src_hbm, dst_hbm, g_out, w_out,
               src_v, dst_v, arow, brow, vrow, sem):
    wid = lax.axis_index("s") * NC + lax.axis_index("c")
    base = wid * EPW

    def chunk(ci, carry):
        off = base + ci * KG
        pltpu.sync_copy(src_hbm.at[pl.ds(off, KG)], src_v)
        pltpu.sync_copy(dst_hbm.at[pl.ds(off, KG)], dst_v)
        pltpu.async_copy(ax_hbm.at[src_v], arow, sem).wait()
        pltpu.async_copy(bx_hbm.at[dst_v], brow, sem).wait()
        pltpu.async_copy(vx_hbm.at[src_v], vrow, sem).wait()

        def row(r, rcarry):
            for c in range(H // 16):
                sl = pl.ds(c * 16, 16)
                arow[r, sl] = arow[r, sl] + brow[r, sl]
            return rcarry
        lax.fori_loop(0, KG, row, 0)
        pltpu.sync_copy(arow, g_out.at[pl.ds(off, KG)])
        pltpu.sync_copy(vrow, w_out.at[pl.ds(off, KG)])
        return carry
    lax.fori_loop(0, EPW // KG, chunk, 0)


HH = H // NC
EPS = E // NS
KS = 80
NP = 10240
RPS = NP // NS
RB = 128


@functools.partial(
    pl.kernel, mesh=_sc_mesh,
    out_type=jax.ShapeDtypeStruct((NP, H), jnp.float32),
    scratch_types=[pltpu.VMEM((KS,), jnp.int32),
                   pltpu.VMEM((KS, HH), jnp.float32),
                   pltpu.VMEM((RB, HH), jnp.float32),
                   pltpu.VMEM_SHARED((NP, HH), jnp.float32),
                   pltpu.SemaphoreType.DMA],
)
def _scatter_sc(p_hbm, dst_hbm, agg_out, dst_v, prow, zbuf, agg_sh, sem):
    cid = lax.axis_index("c")
    sid = lax.axis_index("s")
    c0 = cid * HH

    def zrow(r, carry):
        for c in range(HH // 16):
            zbuf[r, pl.ds(c * 16, 16)] = jnp.zeros((16,), jnp.float32)
        return carry
    lax.fori_loop(0, RB, zrow, 0)
    for j in range(RPS // RB):
        pltpu.sync_copy(zbuf, agg_sh.at[pl.ds(sid * RPS + j * RB, RB)])
    plsc.subcore_barrier()

    def chunk(ci, carry):
        off = sid * EPS + ci * KS
        pltpu.sync_copy(dst_hbm.at[pl.ds(off, KS)], dst_v)
        pltpu.sync_copy(p_hbm.at[pl.ds(off, KS), pl.ds(c0, HH)], prow)
        pltpu.sync_copy(prow, agg_sh.at[dst_v], add=True)
        return carry
    lax.fori_loop(0, EPS // KS, chunk, 0)
    plsc.subcore_barrier()

    for j in range(RPS // RB):
        r0 = sid * RPS + j * RB
        pltpu.sync_copy(agg_sh.at[pl.ds(r0, RB)], zbuf)
        pltpu.sync_copy(zbuf, agg_out.at[pl.ds(r0, RB), pl.ds(c0, HH)])


def kernel(x, e, edge_index, node_W, node_b, edge_W, edge_b,
           Wu, bu, Wv, bv, Wa, ba, Wb, bb, Wc, bc,
           ln_x_g, ln_x_b, ln_e_g, ln_e_b, out_W, out_b):
    src = edge_index[0]
    dst = edge_index[1]
    W4 = jnp.concatenate([Wu, Wv, Wa, Wb], axis=2)
    b4 = jnp.concatenate([bu, bv, ba, bb], axis=1)

    h = _embed(x, node_W, node_b)
    e2 = e.reshape(E, 1)
    f = None
    e_out = None
    for i in range(L):
        Ux, Vx, Ax, Bx = _node_mm(h, W4[i], b4[i])
        G, W = _gather_sc(Ax, Bx, Vx, src, dst)
        if i == 0:
            f, P = _edge_first(e2, edge_W, edge_b, G, W,
                               Wc[i], bc[i], ln_e_g[i], ln_e_b[i])
        elif i < L - 1:
            f, P = _edge_mid(f, G, W, Wc[i], bc[i], ln_e_g[i], ln_e_b[i])
        else:
            P, e_out = _edge_last(f, G, W, Wc[i], bc[i],
                                  ln_e_g[i], ln_e_b[i], out_W, out_b)
        agg = _scatter_sc(P, dst)
        h = _h_update(h, Ux, agg, ln_x_g[i], ln_x_b[i])
    return h, e_out


# speedup vs baseline: 2.4607x; 1.2080x over previous
import functools

import jax
import jax.numpy as jnp
from jax import lax
from jax.experimental import pallas as pl
from jax.experimental.pallas import tpu as pltpu
from jax.experimental.pallas import tpu_sc as plsc

N = 10000
E = 320000
H = 256
L = 12
EB = 1280
NB = 2000

_INTERP = False


def _ln_relu(t, g, b):
    mu = jnp.mean(t, axis=-1, keepdims=True)
    var = jnp.mean((t - mu) ** 2, axis=-1, keepdims=True)
    ln = (t - mu) * lax.rsqrt(var + 1e-5) * g + b
    return jnp.maximum(ln, 0.0)


def _embed_body(x_ref, w_ref, b_ref, o_ref):
    o_ref[...] = jnp.dot(x_ref[...], w_ref[...],
                         preferred_element_type=jnp.float32) + b_ref[...]


def _embed(x, node_W, node_b):
    return pl.pallas_call(
        _embed_body,
        grid=(N // NB,),
        in_specs=[pl.BlockSpec((NB, 2), lambda i: (i, 0)),
                  pl.BlockSpec((2, H), lambda i: (0, 0)),
                  pl.BlockSpec((1, H), lambda i: (0, 0))],
        out_specs=pl.BlockSpec((NB, H), lambda i: (i, 0)),
        out_shape=jax.ShapeDtypeStruct((N, H), jnp.float32),
        interpret=_INTERP,
    )(x, node_W, node_b.reshape(1, H))


HW = H // 2


def _pack_bf16(r):
    bl = lax.bitcast_convert_type(r[:, :HW].astype(jnp.bfloat16),
                                  jnp.uint16).astype(jnp.uint32)
    br = lax.bitcast_convert_type(r[:, HW:].astype(jnp.bfloat16),
                                  jnp.uint16).astype(jnp.uint32)
    return lax.bitcast_convert_type(bl | (br << 16), jnp.int32)


def _unpack_bf16(g):
    gu = lax.bitcast_convert_type(g, jnp.uint32)
    lo = lax.bitcast_convert_type(gu << 16, jnp.float32)
    hi = lax.bitcast_convert_type(gu & jnp.uint32(0xFFFF0000), jnp.float32)
    return jnp.concatenate([lo, hi], axis=-1)


def _node_mm_body(h_ref, w4_ref, b4_ref, u_ref, v_ref, a_ref, b_ref):
    hb = h_ref[...]
    w4 = w4_ref[...]
    outs = (u_ref, v_ref, a_ref, b_ref)
    for j in range(4):
        r = (jnp.dot(hb, w4[:, j * H:(j + 1) * H],
                     preferred_element_type=jnp.float32)
             + b4_ref[0, j * H:(j + 1) * H][None, :])
        outs[j][...] = r if j == 0 else _pack_bf16(r)


def _node_mm(h, W4, b4):
    return pl.pallas_call(
        _node_mm_body,
        grid=(N // NB,),
        in_specs=[pl.BlockSpec((NB, H), lambda i: (i, 0)),
                  pl.BlockSpec((H, 4 * H), lambda i: (0, 0)),
                  pl.BlockSpec((1, 4 * H), lambda i: (0, 0))],
        out_specs=[pl.BlockSpec((NB, H), lambda i: (i, 0))]
                  + [pl.BlockSpec((NB, HW), lambda i: (i, 0))] * 3,
        out_shape=[jax.ShapeDtypeStruct((N, H), jnp.float32)]
                  + [jax.ShapeDtypeStruct((N, HW), jnp.int32)] * 3,
        interpret=_INTERP,
    )(h, W4, b4.reshape(1, 4 * H))


def _edge_core(f, g_ref, w_ref, wc_ref, bc_ref, lng_ref, lnb_ref):
    ga_ref, gb_ref = g_ref
    e_new = (_unpack_bf16(ga_ref[...]) + _unpack_bf16(gb_ref[...])
             + jnp.dot(f, wc_ref[...], preferred_element_type=jnp.float32)
             + bc_ref[...])
    gates = jax.nn.sigmoid(e_new)
    f_new = f + _ln_relu(e_new, lng_ref[...], lnb_ref[...])
    p = gates * _unpack_bf16(w_ref[...])
    return f_new, p


def _edge_first_body(e_ref, ew_ref, eb_ref, ga_ref, gb_ref, w_ref, wc_ref,
                     bc_ref, lng_ref, lnb_ref, fo_ref, p_ref):
    f = e_ref[...] * ew_ref[...] + eb_ref[...]
    f_new, p = _edge_core(f, (ga_ref, gb_ref), w_ref, wc_ref, bc_ref,
                          lng_ref, lnb_ref)
    fo_ref[...] = f_new
    p_ref[...] = p


def _edge_mid_body(f_ref, ga_ref, gb_ref, w_ref, wc_ref, bc_ref, lng_ref,
                   lnb_ref, fo_ref, p_ref):
    f_new, p = _edge_core(f_ref[...], (ga_ref, gb_ref), w_ref, wc_ref, bc_ref,
                          lng_ref, lnb_ref)
    fo_ref[...] = f_new
    p_ref[...] = p


def _edge_last_body(f_ref, ga_ref, gb_ref, w_ref, wc_ref, bc_ref, lng_ref,
                    lnb_ref, ow_ref, ob_ref, p_ref, eo_ref):
    f_new, p = _edge_core(f_ref[...], (ga_ref, gb_ref), w_ref, wc_ref, bc_ref,
                          lng_ref, lnb_ref)
    p_ref[...] = p
    eo_ref[...] = jnp.dot(f_new, ow_ref[...],
                          preferred_element_type=jnp.float32) + ob_ref[...]


_eb_spec = lambda: pl.BlockSpec((EB, H), lambda i: (i, 0))
_ebp_spec = lambda: pl.BlockSpec((EB, HW), lambda i: (i, 0))
_w_spec = lambda: pl.BlockSpec((H, H), lambda i: (0, 0))
_row_spec = lambda: pl.BlockSpec((1, H), lambda i: (0, 0))


def _edge_first(e2, edge_W, edge_b, GA, GB, W, wc, bc, lng, lnb):
    return pl.pallas_call(
        _edge_first_body,
        grid=(E // EB,),
        in_specs=[pl.BlockSpec((EB, 1), lambda i: (i, 0)),
                  _row_spec(), _row_spec(),
                  _ebp_spec(), _ebp_spec(), _ebp_spec(), _w_spec(),
                  _row_spec(), _row_spec(), _row_spec()],
        out_specs=[_eb_spec(), _eb_spec()],
        out_shape=[jax.ShapeDtypeStruct((E, H), jnp.float32)] * 2,
        interpret=_INTERP,
    )(e2, edge_W, edge_b.reshape(1, H), GA, GB, W, wc,
      bc.reshape(1, H), lng.reshape(1, H), lnb.reshape(1, H))


def _edge_mid(f, GA, GB, W, wc, bc, lng, lnb):
    return pl.pallas_call(
        _edge_mid_body,
        grid=(E // EB,),
        in_specs=[_eb_spec(), _ebp_spec(), _ebp_spec(), _ebp_spec(), _w_spec(),
                  _row_spec(), _row_spec(), _row_spec()],
        out_specs=[_eb_spec(), _eb_spec()],
        out_shape=[jax.ShapeDtypeStruct((E, H), jnp.float32)] * 2,
        interpret=_INTERP,
    )(f, GA, GB, W, wc, bc.reshape(1, H), lng.reshape(1, H), lnb.reshape(1, H))


def _edge_last(f, GA, GB, W, wc, bc, lng, lnb, out_W, out_b):
    return pl.pallas_call(
        _edge_last_body,
        grid=(E // EB,),
        in_specs=[_eb_spec(), _ebp_spec(), _ebp_spec(), _ebp_spec(), _w_spec(),
                  _row_spec(), _row_spec(), _row_spec(),
                  pl.BlockSpec((H, 2), lambda i: (0, 0)),
                  pl.BlockSpec((1, 2), lambda i: (0, 0))],
        out_specs=[_eb_spec(), pl.BlockSpec((EB, 2), lambda i: (i, 0))],
        out_shape=[jax.ShapeDtypeStruct((E, H), jnp.float32),
                   jax.ShapeDtypeStruct((E, 2), jnp.float32)],
        interpret=_INTERP,
    )(f, GA, GB, W, wc, bc.reshape(1, H), lng.reshape(1, H), lnb.reshape(1, H),
      out_W, out_b.reshape(1, 2))


def _h_update_body(h_ref, ux_ref, agg_ref, g_ref, b_ref, o_ref):
    t = ux_ref[...] + agg_ref[...]
    o_ref[...] = h_ref[...] + _ln_relu(t, g_ref[...], b_ref[...])


def _h_update(h, Ux, agg, lng, lnb):
    return pl.pallas_call(
        _h_update_body,
        grid=(N // NB,),
        in_specs=[pl.BlockSpec((NB, H), lambda i: (i, 0))] * 3
                 + [_row_spec(), _row_spec()],
        out_specs=pl.BlockSpec((NB, H), lambda i: (i, 0)),
        out_shape=jax.ShapeDtypeStruct((N, H), jnp.float32),
        interpret=_INTERP,
    )(h, Ux, agg, lng.reshape(1, H), lnb.reshape(1, H))


NC = 2
NS = 16
NW = NC * NS

_sc_mesh = plsc.VectorSubcoreMesh(core_axis_name="c", subcore_axis_name="s")

EPW = E // NW
KG = 80


@functools.partial(
    pl.kernel, mesh=_sc_mesh,
    out_type=[jax.ShapeDtypeStruct((E, HW), jnp.int32)] * 3,
    scratch_types=[pltpu.VMEM((KG,), jnp.int32),
                   pltpu.VMEM((KG,), jnp.int32),
                   pltpu.VMEM((KG, HW), jnp.int32),
                   pltpu.VMEM((KG, HW), jnp.int32),
                   pltpu.VMEM((KG, HW), jnp.int32),
                   pltpu.SemaphoreType.DMA],
)
def _gather_sc(ax_hbm, bx_hbm, vx_hbm, src_hbm, dst_hbm, a_out, b_out, v_out,
               src_v, dst_v, arow, brow, vrow, sem):
    wid = lax.axis_index("s") * NC + lax.axis_index("c")
    base = wid * EPW

    def chunk(ci, carry):
        off = base + ci * KG
        pltpu.sync_copy(src_hbm.at[pl.ds(off, KG)], src_v)
        pltpu.sync_copy(dst_hbm.at[pl.ds(off, KG)], dst_v)
        pltpu.async_copy(ax_hbm.at[src_v], arow, sem).wait()
        pltpu.async_copy(bx_hbm.at[dst_v], brow, sem).wait()
        pltpu.async_copy(vx_hbm.at[src_v], vrow, sem).wait()
        pltpu.sync_copy(arow, a_out.at[pl.ds(off, KG)])
        pltpu.sync_copy(brow, b_out.at[pl.ds(off, KG)])
        pltpu.sync_copy(vrow, v_out.at[pl.ds(off, KG)])
        return carry
    lax.fori_loop(0, EPW // KG, chunk, 0)


HH = H // NC
EPS = E // NS
KS = 80
NP = 10240
RPS = NP // NS
RB = 128


@functools.partial(
    pl.kernel, mesh=_sc_mesh,
    out_type=jax.ShapeDtypeStruct((NP, H), jnp.float32),
    scratch_types=[pltpu.VMEM((KS,), jnp.int32),
                   pltpu.VMEM((KS, HH), jnp.float32),
                   pltpu.VMEM((RB, HH), jnp.float32),
                   pltpu.VMEM_SHARED((NP, HH), jnp.float32),
                   pltpu.SemaphoreType.DMA],
)
def _scatter_sc(p_hbm, dst_hbm, agg_out, dst_v, prow, zbuf, agg_sh, sem):
    cid = lax.axis_index("c")
    sid = lax.axis_index("s")
    c0 = cid * HH

    def zrow(r, carry):
        for c in range(HH // 16):
            zbuf[r, pl.ds(c * 16, 16)] = jnp.zeros((16,), jnp.float32)
        return carry
    lax.fori_loop(0, RB, zrow, 0)
    for j in range(RPS // RB):
        pltpu.sync_copy(zbuf, agg_sh.at[pl.ds(sid * RPS + j * RB, RB)])
    plsc.subcore_barrier()

    def chunk(ci, carry):
        off = sid * EPS + ci * KS
        pltpu.sync_copy(dst_hbm.at[pl.ds(off, KS)], dst_v)
        pltpu.sync_copy(p_hbm.at[pl.ds(off, KS), pl.ds(c0, HH)], prow)
        pltpu.sync_copy(prow, agg_sh.at[dst_v], add=True)
        return carry
    lax.fori_loop(0, EPS // KS, chunk, 0)
    plsc.subcore_barrier()

    for j in range(RPS // RB):
        r0 = sid * RPS + j * RB
        pltpu.sync_copy(agg_sh.at[pl.ds(r0, RB)], zbuf)
        pltpu.sync_copy(zbuf, agg_out.at[pl.ds(r0, RB), pl.ds(c0, HH)])


def kernel(x, e, edge_index, node_W, node_b, edge_W, edge_b,
           Wu, bu, Wv, bv, Wa, ba, Wb, bb, Wc, bc,
           ln_x_g, ln_x_b, ln_e_g, ln_e_b, out_W, out_b):
    src = edge_index[0]
    dst = edge_index[1]
    W4 = jnp.concatenate([Wu, Wv, Wa, Wb], axis=2)
    b4 = jnp.concatenate([bu, bv, ba, bb], axis=1)

    h = _embed(x, node_W, node_b)
    e2 = e.reshape(E, 1)
    f = None
    e_out = None
    for i in range(L):
        Ux, Vx, Ax, Bx = _node_mm(h, W4[i], b4[i])
        GA, GB, W = _gather_sc(Ax, Bx, Vx, src, dst)
        if i == 0:
            f, P = _edge_first(e2, edge_W, edge_b, GA, GB, W,
                               Wc[i], bc[i], ln_e_g[i], ln_e_b[i])
        elif i < L - 1:
            f, P = _edge_mid(f, GA, GB, W, Wc[i], bc[i], ln_e_g[i], ln_e_b[i])
        else:
            P, e_out = _edge_last(f, GA, GB, W, Wc[i], bc[i],
                                  ln_e_g[i], ln_e_b[i], out_W, out_b)
        agg = _scatter_sc(P, dst)
        h = _h_update(h, Ux, agg, ln_x_g[i], ln_x_b[i])
    return h, e_out


# speedup vs baseline: 2.8258x; 1.1484x over previous
import functools

import jax
import jax.numpy as jnp
from jax import lax
from jax.experimental import pallas as pl
from jax.experimental.pallas import tpu as pltpu
from jax.experimental.pallas import tpu_sc as plsc

N = 10000
E = 320000
H = 256
L = 12
EB = 1280
NB = 2000

_INTERP = False


def _ln_relu(t, g, b):
    mu = jnp.mean(t, axis=-1, keepdims=True)
    var = jnp.mean((t - mu) ** 2, axis=-1, keepdims=True)
    ln = (t - mu) * lax.rsqrt(var + 1e-5) * g + b
    return jnp.maximum(ln, 0.0)


def _embed_body(x_ref, w_ref, b_ref, o_ref):
    o_ref[...] = jnp.dot(x_ref[...], w_ref[...],
                         preferred_element_type=jnp.float32) + b_ref[...]


def _embed(x, node_W, node_b):
    return pl.pallas_call(
        _embed_body,
        grid=(N // NB,),
        in_specs=[pl.BlockSpec((NB, 2), lambda i: (i, 0)),
                  pl.BlockSpec((2, H), lambda i: (0, 0)),
                  pl.BlockSpec((1, H), lambda i: (0, 0))],
        out_specs=pl.BlockSpec((NB, H), lambda i: (i, 0)),
        out_shape=jax.ShapeDtypeStruct((N, H), jnp.float32),
        interpret=_INTERP,
    )(x, node_W, node_b.reshape(1, H))


HW = H // 2


def _pack_bf16(r):
    bl = lax.bitcast_convert_type(r[:, :HW].astype(jnp.bfloat16),
                                  jnp.uint16).astype(jnp.uint32)
    br = lax.bitcast_convert_type(r[:, HW:].astype(jnp.bfloat16),
                                  jnp.uint16).astype(jnp.uint32)
    return lax.bitcast_convert_type(bl | (br << 16), jnp.int32)


def _unpack_bf16(g):
    gu = lax.bitcast_convert_type(g, jnp.uint32)
    lo = lax.bitcast_convert_type(gu << 16, jnp.float32)
    hi = lax.bitcast_convert_type(gu & jnp.uint32(0xFFFF0000), jnp.float32)
    return jnp.concatenate([lo, hi], axis=-1)


def _node_mm_body(h_ref, w4_ref, b4_ref, u_ref, v_ref, a_ref, b_ref):
    hb = h_ref[...]
    w4 = w4_ref[...]
    outs = (u_ref, v_ref, a_ref, b_ref)
    for j in range(4):
        r = (jnp.dot(hb, w4[:, j * H:(j + 1) * H],
                     preferred_element_type=jnp.float32)
             + b4_ref[0, j * H:(j + 1) * H][None, :])
        outs[j][...] = r if j == 0 else _pack_bf16(r)


def _node_mm(h, W4, b4):
    return pl.pallas_call(
        _node_mm_body,
        grid=(N // NB,),
        in_specs=[pl.BlockSpec((NB, H), lambda i: (i, 0)),
                  pl.BlockSpec((H, 4 * H), lambda i: (0, 0)),
                  pl.BlockSpec((1, 4 * H), lambda i: (0, 0))],
        out_specs=[pl.BlockSpec((NB, H), lambda i: (i, 0))]
                  + [pl.BlockSpec((NB, HW), lambda i: (i, 0))] * 3,
        out_shape=[jax.ShapeDtypeStruct((N, H), jnp.float32)]
                  + [jax.ShapeDtypeStruct((N, HW), jnp.int32)] * 3,
        interpret=_INTERP,
    )(h, W4, b4.reshape(1, 4 * H))


def _edge_core(f, g_ref, w_ref, wc_ref, bc_ref, lng_ref, lnb_ref):
    ga_ref, gb_ref = g_ref
    e_new = (_unpack_bf16(ga_ref[...]) + _unpack_bf16(gb_ref[...])
             + jnp.dot(f, wc_ref[...], preferred_element_type=jnp.float32)
             + bc_ref[...])
    gates = jax.nn.sigmoid(e_new)
    f_new = f + _ln_relu(e_new, lng_ref[...], lnb_ref[...])
    p = gates * _unpack_bf16(w_ref[...])
    return f_new, p


def _store_p(p_ref, p):
    p_ref[0] = p[:, :H // 2]
    p_ref[1] = p[:, H // 2:]


def _edge_first_body(e_ref, ew_ref, eb_ref, ga_ref, gb_ref, w_ref, wc_ref,
                     bc_ref, lng_ref, lnb_ref, fo_ref, p_ref):
    f = e_ref[...] * ew_ref[...] + eb_ref[...]
    f_new, p = _edge_core(f, (ga_ref, gb_ref), w_ref, wc_ref, bc_ref,
                          lng_ref, lnb_ref)
    fo_ref[...] = f_new
    _store_p(p_ref, p)


def _edge_mid_body(f_ref, ga_ref, gb_ref, w_ref, wc_ref, bc_ref, lng_ref,
                   lnb_ref, fo_ref, p_ref):
    f_new, p = _edge_core(f_ref[...], (ga_ref, gb_ref), w_ref, wc_ref, bc_ref,
                          lng_ref, lnb_ref)
    fo_ref[...] = f_new
    _store_p(p_ref, p)


def _edge_last_body(f_ref, ga_ref, gb_ref, w_ref, wc_ref, bc_ref, lng_ref,
                    lnb_ref, ow_ref, ob_ref, p_ref, eo_ref):
    f_new, p = _edge_core(f_ref[...], (ga_ref, gb_ref), w_ref, wc_ref, bc_ref,
                          lng_ref, lnb_ref)
    _store_p(p_ref, p)
    eo_ref[...] = jnp.dot(f_new, ow_ref[...],
                          preferred_element_type=jnp.float32) + ob_ref[...]


_eb_spec = lambda: pl.BlockSpec((EB, H), lambda i: (i, 0))
_ebp_spec = lambda: pl.BlockSpec((EB, HW), lambda i: (i, 0))
_pc_spec = lambda: pl.BlockSpec((2, EB, H // 2), lambda i: (0, i, 0))
_pc_shape = lambda: jax.ShapeDtypeStruct((2, E, H // 2), jnp.float32)
_w_spec = lambda: pl.BlockSpec((H, H), lambda i: (0, 0))
_row_spec = lambda: pl.BlockSpec((1, H), lambda i: (0, 0))


def _edge_first(e2, edge_W, edge_b, GA, GB, W, wc, bc, lng, lnb):
    return pl.pallas_call(
        _edge_first_body,
        grid=(E // EB,),
        in_specs=[pl.BlockSpec((EB, 1), lambda i: (i, 0)),
                  _row_spec(), _row_spec(),
                  _ebp_spec(), _ebp_spec(), _ebp_spec(), _w_spec(),
                  _row_spec(), _row_spec(), _row_spec()],
        out_specs=[_eb_spec(), _pc_spec()],
        out_shape=[jax.ShapeDtypeStruct((E, H), jnp.float32), _pc_shape()],
        interpret=_INTERP,
    )(e2, edge_W, edge_b.reshape(1, H), GA, GB, W, wc,
      bc.reshape(1, H), lng.reshape(1, H), lnb.reshape(1, H))


def _edge_mid(f, GA, GB, W, wc, bc, lng, lnb):
    return pl.pallas_call(
        _edge_mid_body,
        grid=(E // EB,),
        in_specs=[_eb_spec(), _ebp_spec(), _ebp_spec(), _ebp_spec(), _w_spec(),
                  _row_spec(), _row_spec(), _row_spec()],
        out_specs=[_eb_spec(), _pc_spec()],
        out_shape=[jax.ShapeDtypeStruct((E, H), jnp.float32), _pc_shape()],
        interpret=_INTERP,
    )(f, GA, GB, W, wc, bc.reshape(1, H), lng.reshape(1, H), lnb.reshape(1, H))


def _edge_last(f, GA, GB, W, wc, bc, lng, lnb, out_W, out_b):
    return pl.pallas_call(
        _edge_last_body,
        grid=(E // EB,),
        in_specs=[_eb_spec(), _ebp_spec(), _ebp_spec(), _ebp_spec(), _w_spec(),
                  _row_spec(), _row_spec(), _row_spec(),
                  pl.BlockSpec((H, 2), lambda i: (0, 0)),
                  pl.BlockSpec((1, 2), lambda i: (0, 0))],
        out_specs=[_pc_spec(), pl.BlockSpec((EB, 2), lambda i: (i, 0))],
        out_shape=[_pc_shape(),
                   jax.ShapeDtypeStruct((E, 2), jnp.float32)],
        interpret=_INTERP,
    )(f, GA, GB, W, wc, bc.reshape(1, H), lng.reshape(1, H), lnb.reshape(1, H),
      out_W, out_b.reshape(1, 2))


def _h_update_body(h_ref, ux_ref, agg_ref, g_ref, b_ref, o_ref):
    t = ux_ref[...] + agg_ref[...]
    o_ref[...] = h_ref[...] + _ln_relu(t, g_ref[...], b_ref[...])


def _h_update(h, Ux, agg, lng, lnb):
    return pl.pallas_call(
        _h_update_body,
        grid=(N // NB,),
        in_specs=[pl.BlockSpec((NB, H), lambda i: (i, 0))] * 3
                 + [_row_spec(), _row_spec()],
        out_specs=pl.BlockSpec((NB, H), lambda i: (i, 0)),
        out_shape=jax.ShapeDtypeStruct((N, H), jnp.float32),
        interpret=_INTERP,
    )(h, Ux, agg, lng.reshape(1, H), lnb.reshape(1, H))


NC = 2
NS = 16
NW = NC * NS

_sc_mesh = plsc.VectorSubcoreMesh(core_axis_name="c", subcore_axis_name="s")

EPW = E // NW
KG = 40


@functools.partial(
    pl.kernel, mesh=_sc_mesh,
    out_type=[jax.ShapeDtypeStruct((E, HW), jnp.int32)] * 3,
    scratch_types=[pltpu.VMEM((2, KG), jnp.int32),
                   pltpu.VMEM((2, KG), jnp.int32),
                   pltpu.VMEM((2, KG, HW), jnp.int32),
                   pltpu.VMEM((2, KG, HW), jnp.int32),
                   pltpu.VMEM((2, KG, HW), jnp.int32),
                   pltpu.SemaphoreType.DMA,
                   pltpu.SemaphoreType.DMA,
                   pltpu.SemaphoreType.DMA],
)
def _gather_sc(ax_hbm, bx_hbm, vx_hbm, src_hbm, dst_hbm, a_out, b_out, v_out,
               src_v, dst_v, arow, brow, vrow, gsem, wsem0, wsem1):
    wid = lax.axis_index("s") * NC + lax.axis_index("c")
    base = wid * EPW
    wsems = (wsem0, wsem1)

    def half(cj, b):
        off = base + (cj * 2 + b) * KG
        ab, bb, vb = arow.at[b], brow.at[b], vrow.at[b]
        wsem = wsems[b]

        @pl.when(cj > 0)
        def _():
            pltpu.make_async_copy(a_out.at[pl.ds(0, KG)], ab, wsem).wait()
            pltpu.make_async_copy(a_out.at[pl.ds(0, KG)], bb, wsem).wait()
            pltpu.make_async_copy(a_out.at[pl.ds(0, KG)], vb, wsem).wait()
        pltpu.sync_copy(src_hbm.at[pl.ds(off, KG)], src_v.at[b])
        pltpu.sync_copy(dst_hbm.at[pl.ds(off, KG)], dst_v.at[b])
        ca = pltpu.async_copy(ax_hbm.at[src_v.at[b]], ab, gsem)
        cb = pltpu.async_copy(bx_hbm.at[dst_v.at[b]], bb, gsem)
        cv = pltpu.async_copy(vx_hbm.at[src_v.at[b]], vb, gsem)
        ca.wait(); cb.wait(); cv.wait()
        pltpu.async_copy(ab, a_out.at[pl.ds(off, KG)], wsem)
        pltpu.async_copy(bb, b_out.at[pl.ds(off, KG)], wsem)
        pltpu.async_copy(vb, v_out.at[pl.ds(off, KG)], wsem)

    def pair(cj, carry):
        half(cj, 0)
        half(cj, 1)
        return carry
    lax.fori_loop(0, EPW // (2 * KG), pair, 0)
    for b in range(2):
        pltpu.make_async_copy(a_out.at[pl.ds(0, KG)], arow.at[b], wsems[b]).wait()
        pltpu.make_async_copy(a_out.at[pl.ds(0, KG)], brow.at[b], wsems[b]).wait()
        pltpu.make_async_copy(a_out.at[pl.ds(0, KG)], vrow.at[b], wsems[b]).wait()


HH = H // NC
EPS = E // NS
KS = 80
NP = 10240
RPS = NP // NS
RB = 128


@functools.partial(
    pl.kernel, mesh=_sc_mesh,
    out_type=jax.ShapeDtypeStruct((NP, H), jnp.float32),
    scratch_types=[pltpu.VMEM((2, KS), jnp.int32),
                   pltpu.VMEM((2, KS, HH), jnp.float32),
                   pltpu.VMEM((RB, HH), jnp.float32),
                   pltpu.VMEM_SHARED((NP, HH), jnp.float32),
                   pltpu.SemaphoreType.DMA,
                   pltpu.SemaphoreType.DMA,
                   pltpu.SemaphoreType.DMA],
)
def _scatter_sc(p_hbm, dst_hbm, agg_out, dst_v, prow, zbuf, agg_sh,
                rsem, asem0, asem1):
    cid = lax.axis_index("c")
    sid = lax.axis_index("s")
    c0 = cid * HH
    asems = (asem0, asem1)

    def zrow(r, carry):
        for c in range(HH // 16):
            zbuf[r, pl.ds(c * 16, 16)] = jnp.zeros((16,), jnp.float32)
        return carry
    lax.fori_loop(0, RB, zrow, 0)
    for j in range(RPS // RB):
        pltpu.sync_copy(zbuf, agg_sh.at[pl.ds(sid * RPS + j * RB, RB)])
    plsc.subcore_barrier()

    def half(cj, b):
        off = sid * EPS + (cj * 2 + b) * KS
        pb = prow.at[b]
        asem = asems[b]

        @pl.when(cj > 0)
        def _():
            pltpu.make_async_copy(p_hbm.at[0, pl.ds(0, KS)], pb, asem).wait()
        pltpu.sync_copy(dst_hbm.at[pl.ds(off, KS)], dst_v.at[b])
        pltpu.async_copy(p_hbm.at[cid, pl.ds(off, KS)], pb, rsem).wait()
        pltpu.async_copy(pb, agg_sh.at[dst_v.at[b]], asem, add=True)

    def pair(cj, carry):
        half(cj, 0)
        half(cj, 1)
        return carry
    lax.fori_loop(0, EPS // (2 * KS), pair, 0)
    for b in range(2):
        pltpu.make_async_copy(p_hbm.at[0, pl.ds(0, KS)], prow.at[b],
                              asems[b]).wait()
    plsc.subcore_barrier()

    for j in range(RPS // RB):
        r0 = sid * RPS + j * RB
        pltpu.sync_copy(agg_sh.at[pl.ds(r0, RB)], zbuf)
        pltpu.sync_copy(zbuf, agg_out.at[pl.ds(r0, RB), pl.ds(c0, HH)])


def kernel(x, e, edge_index, node_W, node_b, edge_W, edge_b,
           Wu, bu, Wv, bv, Wa, ba, Wb, bb, Wc, bc,
           ln_x_g, ln_x_b, ln_e_g, ln_e_b, out_W, out_b):
    src = edge_index[0]
    dst = edge_index[1]
    W4 = jnp.concatenate([Wu, Wv, Wa, Wb], axis=2)
    b4 = jnp.concatenate([bu, bv, ba, bb], axis=1)

    h = _embed(x, node_W, node_b)
    e2 = e.reshape(E, 1)
    f = None
    e_out = None
    for i in range(L):
        Ux, Vx, Ax, Bx = _node_mm(h, W4[i], b4[i])
        GA, GB, W = _gather_sc(Ax, Bx, Vx, src, dst)
        if i == 0:
            f, P = _edge_first(e2, edge_W, edge_b, GA, GB, W,
                               Wc[i], bc[i], ln_e_g[i], ln_e_b[i])
        elif i < L - 1:
            f, P = _edge_mid(f, GA, GB, W, Wc[i], bc[i], ln_e_g[i], ln_e_b[i])
        else:
            P, e_out = _edge_last(f, GA, GB, W, Wc[i], bc[i],
                                  ln_e_g[i], ln_e_b[i], out_W, out_b)
        agg = _scatter_sc(P, dst)
        h = _h_update(h, Ux, agg, ln_x_g[i], ln_x_b[i])
    return h, e_out


# speedup vs baseline: 3.2824x; 1.1616x over previous
import functools

import jax
import jax.numpy as jnp
from jax import lax
from jax.experimental import pallas as pl
from jax.experimental.pallas import tpu as pltpu
from jax.experimental.pallas import tpu_sc as plsc

N = 10000
E = 320000
H = 256
L = 12
EB = 1280
NB = 2000

_INTERP = False


def _ln_relu(t, g, b):
    mu = jnp.mean(t, axis=-1, keepdims=True)
    var = jnp.mean((t - mu) ** 2, axis=-1, keepdims=True)
    ln = (t - mu) * lax.rsqrt(var + 1e-5) * g + b
    return jnp.maximum(ln, 0.0)


def _embed_body(x_ref, w_ref, b_ref, o_ref):
    o_ref[...] = jnp.dot(x_ref[...], w_ref[...],
                         preferred_element_type=jnp.float32) + b_ref[...]


def _embed(x, node_W, node_b):
    return pl.pallas_call(
        _embed_body,
        grid=(N // NB,),
        in_specs=[pl.BlockSpec((NB, 2), lambda i: (i, 0)),
                  pl.BlockSpec((2, H), lambda i: (0, 0)),
                  pl.BlockSpec((1, H), lambda i: (0, 0))],
        out_specs=pl.BlockSpec((NB, H), lambda i: (i, 0)),
        out_shape=jax.ShapeDtypeStruct((N, H), jnp.float32),
        interpret=_INTERP,
    )(x, node_W, node_b.reshape(1, H))


HW = H // 2


def _pack_bf16(r):
    bl = lax.bitcast_convert_type(r[:, :HW].astype(jnp.bfloat16),
                                  jnp.uint16).astype(jnp.uint32)
    br = lax.bitcast_convert_type(r[:, HW:].astype(jnp.bfloat16),
                                  jnp.uint16).astype(jnp.uint32)
    return lax.bitcast_convert_type(bl | (br << 16), jnp.int32)


def _unpack_bf16(g):
    gu = lax.bitcast_convert_type(g, jnp.uint32)
    lo = lax.bitcast_convert_type(gu << 16, jnp.float32)
    hi = lax.bitcast_convert_type(gu & jnp.uint32(0xFFFF0000), jnp.float32)
    return jnp.concatenate([lo, hi], axis=-1)


def _node_mm_body(h_ref, w4_ref, b4_ref, u_ref, v_ref, a_ref, b_ref):
    hb = h_ref[...]
    w4 = w4_ref[...]
    outs = (u_ref, v_ref, a_ref, b_ref)
    for j in range(4):
        r = (jnp.dot(hb, w4[:, j * H:(j + 1) * H],
                     preferred_element_type=jnp.float32)
             + b4_ref[0, j * H:(j + 1) * H][None, :])
        outs[j][...] = r if j == 0 else _pack_bf16(r)


def _node_mm(h, W4, b4):
    return pl.pallas_call(
        _node_mm_body,
        grid=(N // NB,),
        in_specs=[pl.BlockSpec((NB, H), lambda i: (i, 0)),
                  pl.BlockSpec((H, 4 * H), lambda i: (0, 0)),
                  pl.BlockSpec((1, 4 * H), lambda i: (0, 0))],
        out_specs=[pl.BlockSpec((NB, H), lambda i: (i, 0))]
                  + [pl.BlockSpec((NB, HW), lambda i: (i, 0))] * 3,
        out_shape=[jax.ShapeDtypeStruct((N, H), jnp.float32)]
                  + [jax.ShapeDtypeStruct((N, HW), jnp.int32)] * 3,
        interpret=_INTERP,
    )(h, W4, b4.reshape(1, 4 * H))


def _edge_core(f, g_ref, w_ref, wc_ref, bc_ref, lng_ref, lnb_ref):
    ga_ref, gb_ref = g_ref
    e_new = (_unpack_bf16(ga_ref[...]) + _unpack_bf16(gb_ref[...])
             + jnp.dot(f, wc_ref[...], preferred_element_type=jnp.float32)
             + bc_ref[...])
    gates = jax.nn.sigmoid(e_new)
    f_new = f + _ln_relu(e_new, lng_ref[...], lnb_ref[...])
    p = gates * _unpack_bf16(w_ref[...])
    return f_new, p


def _store_p(p_ref, p):
    p_ref[0] = p[:, :H // 2]
    p_ref[1] = p[:, H // 2:]


def _edge_first_body(e_ref, ew_ref, eb_ref, ga_ref, gb_ref, w_ref, wc_ref,
                     bc_ref, lng_ref, lnb_ref, fo_ref, p_ref):
    f = e_ref[...] * ew_ref[...] + eb_ref[...]
    f_new, p = _edge_core(f, (ga_ref, gb_ref), w_ref, wc_ref, bc_ref,
                          lng_ref, lnb_ref)
    fo_ref[...] = f_new
    _store_p(p_ref, p)


def _edge_mid_body(f_ref, ga_ref, gb_ref, w_ref, wc_ref, bc_ref, lng_ref,
                   lnb_ref, fo_ref, p_ref):
    f_new, p = _edge_core(f_ref[...], (ga_ref, gb_ref), w_ref, wc_ref, bc_ref,
                          lng_ref, lnb_ref)
    fo_ref[...] = f_new
    _store_p(p_ref, p)


def _edge_last_body(f_ref, ga_ref, gb_ref, w_ref, wc_ref, bc_ref, lng_ref,
                    lnb_ref, ow_ref, ob_ref, p_ref, eo_ref):
    f_new, p = _edge_core(f_ref[...], (ga_ref, gb_ref), w_ref, wc_ref, bc_ref,
                          lng_ref, lnb_ref)
    _store_p(p_ref, p)
    eo_ref[...] = jnp.dot(f_new, ow_ref[...],
                          preferred_element_type=jnp.float32) + ob_ref[...]


_eb_spec = lambda: pl.BlockSpec((EB, H), lambda i: (i, 0))
_ebp_spec = lambda: pl.BlockSpec((EB, HW), lambda i: (i, 0))
_pc_spec = lambda: pl.BlockSpec((2, EB, H // 2), lambda i: (0, i, 0))
_pc_shape = lambda: jax.ShapeDtypeStruct((2, EH, H // 2), jnp.float32)
_w_spec = lambda: pl.BlockSpec((H, H), lambda i: (0, 0))
_row_spec = lambda: pl.BlockSpec((1, H), lambda i: (0, 0))


def _edge_first(e2, edge_W, edge_b, GA, GB, W, wc, bc, lng, lnb):
    return pl.pallas_call(
        _edge_first_body,
        grid=(EH // EB,),
        in_specs=[pl.BlockSpec((EB, 1), lambda i: (i, 0)),
                  _row_spec(), _row_spec(),
                  _ebp_spec(), _ebp_spec(), _ebp_spec(), _w_spec(),
                  _row_spec(), _row_spec(), _row_spec()],
        out_specs=[_eb_spec(), _pc_spec()],
        out_shape=[jax.ShapeDtypeStruct((EH, H), jnp.float32), _pc_shape()],
        interpret=_INTERP,
    )(e2, edge_W, edge_b.reshape(1, H), GA, GB, W, wc,
      bc.reshape(1, H), lng.reshape(1, H), lnb.reshape(1, H))


def _edge_mid(f, GA, GB, W, wc, bc, lng, lnb):
    return pl.pallas_call(
        _edge_mid_body,
        grid=(EH // EB,),
        in_specs=[_eb_spec(), _ebp_spec(), _ebp_spec(), _ebp_spec(), _w_spec(),
                  _row_spec(), _row_spec(), _row_spec()],
        out_specs=[_eb_spec(), _pc_spec()],
        out_shape=[jax.ShapeDtypeStruct((EH, H), jnp.float32), _pc_shape()],
        interpret=_INTERP,
    )(f, GA, GB, W, wc, bc.reshape(1, H), lng.reshape(1, H), lnb.reshape(1, H))


def _edge_last(f, GA, GB, W, wc, bc, lng, lnb, out_W, out_b):
    return pl.pallas_call(
        _edge_last_body,
        grid=(EH // EB,),
        in_specs=[_eb_spec(), _ebp_spec(), _ebp_spec(), _ebp_spec(), _w_spec(),
                  _row_spec(), _row_spec(), _row_spec(),
                  pl.BlockSpec((H, 2), lambda i: (0, 0)),
                  pl.BlockSpec((1, 2), lambda i: (0, 0))],
        out_specs=[_pc_spec(), pl.BlockSpec((EB, 2), lambda i: (i, 0))],
        out_shape=[_pc_shape(),
                   jax.ShapeDtypeStruct((EH, 2), jnp.float32)],
        interpret=_INTERP,
    )(f, GA, GB, W, wc, bc.reshape(1, H), lng.reshape(1, H), lnb.reshape(1, H),
      out_W, out_b.reshape(1, 2))


def _h_update_body(h_ref, ux_ref, agg1_ref, agg2_ref, g_ref, b_ref, o_ref):
    t = ux_ref[...] + agg1_ref[...] + agg2_ref[...]
    o_ref[...] = h_ref[...] + _ln_relu(t, g_ref[...], b_ref[...])


def _h_update(h, Ux, agg1, agg2, lng, lnb):
    return pl.pallas_call(
        _h_update_body,
        grid=(N // NB,),
        in_specs=[pl.BlockSpec((NB, H), lambda i: (i, 0))] * 4
                 + [_row_spec(), _row_spec()],
        out_specs=pl.BlockSpec((NB, H), lambda i: (i, 0)),
        out_shape=jax.ShapeDtypeStruct((N, H), jnp.float32),
        interpret=_INTERP,
    )(h, Ux, agg1, agg2, lng.reshape(1, H), lnb.reshape(1, H))


NC = 2
NS = 16
NW = NC * NS

_sc_mesh = plsc.VectorSubcoreMesh(core_axis_name="c", subcore_axis_name="s")

EH = E // 2
EPW = EH // NW
KG = 40
NCHG = EPW // KG


@functools.partial(
    pl.kernel, mesh=_sc_mesh,
    out_type=[jax.ShapeDtypeStruct((EH, HW), jnp.int32)] * 3,
    scratch_types=[pltpu.VMEM((2, KG), jnp.int32),
                   pltpu.VMEM((2, KG), jnp.int32),
                   pltpu.VMEM((2, KG, HW), jnp.int32),
                   pltpu.VMEM((2, KG, HW), jnp.int32),
                   pltpu.VMEM((2, KG, HW), jnp.int32),
                   pltpu.SemaphoreType.DMA,
                   pltpu.SemaphoreType.DMA,
                   pltpu.SemaphoreType.DMA],
)
def _gather_sc(ax_hbm, bx_hbm, vx_hbm, src_hbm, dst_hbm, a_out, b_out, v_out,
               src_v, dst_v, arow, brow, vrow, gsem, wsem0, wsem1):
    wid = lax.axis_index("s") * NC + lax.axis_index("c")
    base = wid * EPW
    wsems = (wsem0, wsem1)

    def half(cj, b):
        off = base + (cj * 2 + b) * KG
        ab, bb, vb = arow.at[b], brow.at[b], vrow.at[b]
        wsem = wsems[b]

        @pl.when(cj > 0)
        def _():
            pltpu.make_async_copy(a_out.at[pl.ds(0, KG)], ab, wsem).wait()
            pltpu.make_async_copy(a_out.at[pl.ds(0, KG)], bb, wsem).wait()
            pltpu.make_async_copy(a_out.at[pl.ds(0, KG)], vb, wsem).wait()
        pltpu.sync_copy(src_hbm.at[pl.ds(off, KG)], src_v.at[b])
        pltpu.sync_copy(dst_hbm.at[pl.ds(off, KG)], dst_v.at[b])
        ca = pltpu.async_copy(ax_hbm.at[src_v.at[b]], ab, gsem)
        cb = pltpu.async_copy(bx_hbm.at[dst_v.at[b]], bb, gsem)
        cv = pltpu.async_copy(vx_hbm.at[src_v.at[b]], vb, gsem)
        ca.wait(); cb.wait(); cv.wait()
        pltpu.async_copy(ab, a_out.at[pl.ds(off, KG)], wsem)
        pltpu.async_copy(bb, b_out.at[pl.ds(off, KG)], wsem)
        pltpu.async_copy(vb, v_out.at[pl.ds(off, KG)], wsem)

    def pair(cj, carry):
        half(cj, 0)
        half(cj, 1)
        return carry
    lax.fori_loop(0, NCHG // 2, pair, 0)
    if NCHG % 2:
        half(NCHG // 2, 0)
    for b in range(2):
        pltpu.make_async_copy(a_out.at[pl.ds(0, KG)], arow.at[b], wsems[b]).wait()
        pltpu.make_async_copy(a_out.at[pl.ds(0, KG)], brow.at[b], wsems[b]).wait()
        pltpu.make_async_copy(a_out.at[pl.ds(0, KG)], vrow.at[b], wsems[b]).wait()


HH = H // NC
EPS = EH // NS
KS = 80
NCHS = EPS // KS
NP = 10240
RPS = NP // NS
RB = 128


@functools.partial(
    pl.kernel, mesh=_sc_mesh,
    out_type=jax.ShapeDtypeStruct((NP, H), jnp.float32),
    scratch_types=[pltpu.VMEM((2, KS), jnp.int32),
                   pltpu.VMEM((2, KS, HH), jnp.float32),
                   pltpu.VMEM((RB, HH), jnp.float32),
                   pltpu.VMEM_SHARED((NP, HH), jnp.float32),
                   pltpu.SemaphoreType.DMA,
                   pltpu.SemaphoreType.DMA,
                   pltpu.SemaphoreType.DMA],
)
def _scatter_sc(p_hbm, dst_hbm, agg_out, dst_v, prow, zbuf, agg_sh,
                rsem, asem0, asem1):
    cid = lax.axis_index("c")
    sid = lax.axis_index("s")
    c0 = cid * HH
    asems = (asem0, asem1)

    def zrow(r, carry):
        for c in range(HH // 16):
            zbuf[r, pl.ds(c * 16, 16)] = jnp.zeros((16,), jnp.float32)
        return carry
    lax.fori_loop(0, RB, zrow, 0)
    for j in range(RPS // RB):
        pltpu.sync_copy(zbuf, agg_sh.at[pl.ds(sid * RPS + j * RB, RB)])
    plsc.subcore_barrier()

    def half(cj, b):
        off = sid * EPS + (cj * 2 + b) * KS
        pb = prow.at[b]
        asem = asems[b]

        @pl.when(cj > 0)
        def _():
            pltpu.make_async_copy(p_hbm.at[0, pl.ds(0, KS)], pb, asem).wait()
        pltpu.sync_copy(dst_hbm.at[pl.ds(off, KS)], dst_v.at[b])
        pltpu.async_copy(p_hbm.at[cid, pl.ds(off, KS)], pb, rsem).wait()
        pltpu.async_copy(pb, agg_sh.at[dst_v.at[b]], asem, add=True)

    def pair(cj, carry):
        half(cj, 0)
        half(cj, 1)
        return carry
    lax.fori_loop(0, NCHS // 2, pair, 0)
    if NCHS % 2:
        half(NCHS // 2, 0)
    for b in range(2):
        pltpu.make_async_copy(p_hbm.at[0, pl.ds(0, KS)], prow.at[b],
                              asems[b]).wait()
    plsc.subcore_barrier()

    for j in range(RPS // RB):
        r0 = sid * RPS + j * RB
        pltpu.sync_copy(agg_sh.at[pl.ds(r0, RB)], zbuf)
        pltpu.sync_copy(zbuf, agg_out.at[pl.ds(r0, RB), pl.ds(c0, HH)])


def kernel(x, e, edge_index, node_W, node_b, edge_W, edge_b,
           Wu, bu, Wv, bv, Wa, ba, Wb, bb, Wc, bc,
           ln_x_g, ln_x_b, ln_e_g, ln_e_b, out_W, out_b):
    src = edge_index[0]
    dst = edge_index[1]
    W4 = jnp.concatenate([Wu, Wv, Wa, Wb], axis=2)
    b4 = jnp.concatenate([bu, bv, ba, bb], axis=1)

    h = _embed(x, node_W, node_b)
    e2 = e.reshape(E, 1)
    srcs = (src[:EH], src[EH:])
    dsts = (dst[:EH], dst[EH:])
    e2s = (e2[:EH], e2[EH:])
    f = [None, None]
    eo = [None, None]
    for i in range(L):
        Ux, Vx, Ax, Bx = _node_mm(h, W4[i], b4[i])
        aggs = [None, None]
        for k in range(2):
            GA, GB, W = _gather_sc(Ax, Bx, Vx, srcs[k], dsts[k])
            if i == 0:
                f[k], P = _edge_first(e2s[k], edge_W, edge_b, GA, GB, W,
                                      Wc[i], bc[i], ln_e_g[i], ln_e_b[i])
            elif i < L - 1:
                f[k], P = _edge_mid(f[k], GA, GB, W, Wc[i], bc[i],
                                    ln_e_g[i], ln_e_b[i])
            else:
                P, eo[k] = _edge_last(f[k], GA, GB, W, Wc[i], bc[i],
                                      ln_e_g[i], ln_e_b[i], out_W, out_b)
            aggs[k] = _scatter_sc(P, dsts[k])
        h = _h_update(h, Ux, aggs[0], aggs[1], ln_x_g[i], ln_x_b[i])
    e_out = jnp.concatenate([eo[0], eo[1]], axis=0)
    return h, e_out
